# baseline (device time: 27722 ns/iter reference)
import jax
import jax.numpy as jnp
from jax import lax
from jax.experimental import pallas as pl
from jax.experimental.pallas import tpu as pltpu

CHUNK_SIZES = (32, 32, 16, 16, 16, 8, 8)
N_CHUNKS = len(CHUNK_SIZES)
CHUNK_OFFS = tuple(sum(CHUNK_SIZES[:i]) for i in range(N_CHUNKS))
GEMM_BLOCKS = ((0, 32, (0,)), (32, 32, (1,)), (64, 64, (2, 3, 4, 5, 6)))


def kernel(x, W):
    t, d = x.shape
    _, v_local = W.shape
    v_global = 2 * v_local
    th = t // 2

    def body(x_ref, w_ref, out_ref, log_ref,
             ysend_sems, yrecv_sems, xsend_sems, xrecv_sems):
        my_x = lax.axis_index("x")
        my_y = lax.axis_index("y")
        ypeer = (my_x, 1 - my_y)
        xpeer = (1 - my_x, my_y)

        barrier_sem = pltpu.get_barrier_semaphore()
        for nbr in (ypeer, xpeer):
            pl.semaphore_signal(
                barrier_sem, inc=1,
                device_id=nbr, device_id_type=pl.DeviceIdType.MESH,
            )

        w_bf16 = w_ref[...].astype(jnp.bfloat16)
        my_col = my_y * v_local
        peer_col = (1 - my_y) * v_local

        def gemm_block(row_start, rows):
            log_ref[pl.ds(row_start, rows), pl.ds(my_col, v_local)] = (
                lax.dot_general(
                    x_ref[pl.ds(row_start, rows)].astype(jnp.bfloat16),
                    w_bf16,
                    (((1,), (0,)), ((), ())),
                    preferred_element_type=jnp.float32,
                ).astype(jnp.bfloat16)
            )

        def softmax_rows(rows):
            l0 = log_ref[rows, :v_local].astype(jnp.float32)
            l1 = log_ref[rows, v_local:].astype(jnp.float32)
            m = jnp.maximum(
                jnp.max(l0, axis=-1, keepdims=True),
                jnp.max(l1, axis=-1, keepdims=True),
            )
            e0 = jnp.exp(l0 - m)
            e1 = jnp.exp(l1 - m)
            s = jnp.sum(e0, axis=-1, keepdims=True) + jnp.sum(
                e1, axis=-1, keepdims=True
            )
            r = 1.0 / s
            out_ref[rows, :v_local] = (e0 * r).astype(jnp.bfloat16)
            out_ref[rows, v_local:] = (e1 * r).astype(jnp.bfloat16)

        yrdmas = [None] * N_CHUNKS
        first = True
        for b_off, b_rows, b_chunks in GEMM_BLOCKS:
            gemm_block(my_x * th + b_off, b_rows)
            if first:
                pl.semaphore_wait(barrier_sem, 2)
                first = False
            for c in b_chunks:
                row = my_x * th + CHUNK_OFFS[c]
                rdma = pltpu.make_async_remote_copy(
                    src_ref=log_ref.at[
                        pl.ds(row, CHUNK_SIZES[c]), pl.ds(my_col, v_local)
                    ],
                    dst_ref=log_ref.at[
                        pl.ds(row, CHUNK_SIZES[c]), pl.ds(my_col, v_local)
                    ],
                    send_sem=ysend_sems.at[c],
                    recv_sem=yrecv_sems.at[c],
                    device_id=ypeer,
                    device_id_type=pl.DeviceIdType.MESH,
                )
                rdma.start()
                yrdmas[c] = rdma

        xrdmas = [None] * N_CHUNKS
        for c in range(N_CHUNKS):
            row = my_x * th + CHUNK_OFFS[c]
            rows = pl.ds(row, CHUNK_SIZES[c])
            yrdmas[c].wait_recv()
            rdma = pltpu.make_async_remote_copy(
                src_ref=log_ref.at[rows, pl.ds(peer_col, v_local)],
                dst_ref=log_ref.at[rows, pl.ds(peer_col, v_local)],
                send_sem=xsend_sems.at[c],
                recv_sem=xrecv_sems.at[c],
                device_id=xpeer,
                device_id_type=pl.DeviceIdType.MESH,
            )
            rdma.start()
            xrdmas[c] = rdma
            if c < 2:
                gemm_block((1 - my_x) * th + c * 64, 64)
            softmax_rows(rows)

        for c in range(N_CHUNKS):
            row = (1 - my_x) * th + CHUNK_OFFS[c]
            xrdmas[c].wait_recv()
            softmax_rows(pl.ds(row, CHUNK_SIZES[c]))

        for c in range(N_CHUNKS):
            yrdmas[c].wait_send()
            xrdmas[c].wait_send()

    return pl.pallas_call(
        body,
        out_shape=jax.ShapeDtypeStruct((t, v_global), jnp.bfloat16),
        in_specs=[
            pl.BlockSpec(memory_space=pltpu.VMEM),
            pl.BlockSpec(memory_space=pltpu.VMEM),
        ],
        out_specs=pl.BlockSpec(memory_space=pltpu.VMEM),
        scratch_shapes=[
            pltpu.VMEM((t, v_global), jnp.bfloat16),
            pltpu.SemaphoreType.DMA((N_CHUNKS,)),
            pltpu.SemaphoreType.DMA((N_CHUNKS,)),
            pltpu.SemaphoreType.DMA((N_CHUNKS,)),
            pltpu.SemaphoreType.DMA((N_CHUNKS,)),
        ],
        compiler_params=pltpu.CompilerParams(collective_id=0),
    )(x, W)


# device time: 27110 ns/iter; 1.0226x vs baseline; 1.0226x over previous
import jax
import jax.numpy as jnp
from jax import lax
from jax.experimental import pallas as pl
from jax.experimental.pallas import tpu as pltpu

N_CHUNKS = 8
GEMM_BLOCK = 64
CHUNKS_PER_BLOCK = GEMM_BLOCK // (128 // N_CHUNKS)


def kernel(x, W):
    t, d = x.shape
    _, v_local = W.shape
    v_global = 2 * v_local
    th = t // 2
    tc = th // N_CHUNKS
    n_blocks = th // GEMM_BLOCK

    def body(x_hbm, w_hbm, out_hbm, xv, wv, log_ref, out_v,
             in_sems, out_sem,
             ysend_sems, yrecv_sems, xsend_sems, xrecv_sems):
        my_x = lax.axis_index("x")
        my_y = lax.axis_index("y")
        ypeer = (my_x, 1 - my_y)
        xpeer = (1 - my_x, my_y)

        wcopy = pltpu.make_async_copy(w_hbm, wv, in_sems.at[0])
        wcopy.start()
        xcopy = pltpu.make_async_copy(x_hbm, xv, in_sems.at[1])
        xcopy.start()

        barrier_sem = pltpu.get_barrier_semaphore()
        for nbr in (ypeer, xpeer):
            pl.semaphore_signal(
                barrier_sem, inc=1,
                device_id=nbr, device_id_type=pl.DeviceIdType.MESH,
            )

        xcopy.wait()
        wcopy.wait()
        w_bf16 = wv[...].astype(jnp.bfloat16)
        my_col = my_y * v_local
        peer_col = (1 - my_y) * v_local

        def gemm_block(row_start):
            log_ref[pl.ds(row_start, GEMM_BLOCK), pl.ds(my_col, v_local)] = (
                lax.dot_general(
                    xv[pl.ds(row_start, GEMM_BLOCK)].astype(jnp.bfloat16),
                    w_bf16,
                    (((1,), (0,)), ((), ())),
                    preferred_element_type=jnp.float32,
                ).astype(jnp.bfloat16)
            )

        def softmax_rows(rows):
            l0 = log_ref[rows, :v_local].astype(jnp.float32)
            l1 = log_ref[rows, v_local:].astype(jnp.float32)
            m = jnp.maximum(
                jnp.max(l0, axis=-1, keepdims=True),
                jnp.max(l1, axis=-1, keepdims=True),
            )
            e0 = jnp.exp(l0 - m)
            e1 = jnp.exp(l1 - m)
            s = jnp.sum(e0, axis=-1, keepdims=True) + jnp.sum(
                e1, axis=-1, keepdims=True
            )
            r = 1.0 / s
            out_v[rows, :v_local] = (e0 * r).astype(jnp.bfloat16)
            out_v[rows, v_local:] = (e1 * r).astype(jnp.bfloat16)
            out_dma = pltpu.make_async_copy(
                out_v.at[rows], out_hbm.at[rows], out_sem
            )
            out_dma.start()
            return out_dma

        yrdmas = []
        for b in range(n_blocks):
            gemm_block(my_x * th + b * GEMM_BLOCK)
            if b == 0:
                pl.semaphore_wait(barrier_sem, 2)
            for i in range(CHUNKS_PER_BLOCK):
                c = b * CHUNKS_PER_BLOCK + i
                row = my_x * th + c * tc
                rdma = pltpu.make_async_remote_copy(
                    src_ref=log_ref.at[pl.ds(row, tc), pl.ds(my_col, v_local)],
                    dst_ref=log_ref.at[pl.ds(row, tc), pl.ds(my_col, v_local)],
                    send_sem=ysend_sems.at[c],
                    recv_sem=yrecv_sems.at[c],
                    device_id=ypeer,
                    device_id_type=pl.DeviceIdType.MESH,
                )
                rdma.start()
                yrdmas.append(rdma)

        xrdmas = []
        out_dmas = []
        for c in range(N_CHUNKS):
            row = my_x * th + c * tc
            rows = pl.ds(row, tc)
            yrdmas[c].wait_recv()
            rdma = pltpu.make_async_remote_copy(
                src_ref=log_ref.at[rows, pl.ds(peer_col, v_local)],
                dst_ref=log_ref.at[rows, pl.ds(peer_col, v_local)],
                send_sem=xsend_sems.at[c],
                recv_sem=xrecv_sems.at[c],
                device_id=xpeer,
                device_id_type=pl.DeviceIdType.MESH,
            )
            rdma.start()
            xrdmas.append(rdma)
            if c < n_blocks:
                gemm_block((1 - my_x) * th + c * GEMM_BLOCK)
            out_dmas.append(softmax_rows(rows))

        for c in range(N_CHUNKS):
            row = (1 - my_x) * th + c * tc
            xrdmas[c].wait_recv()
            out_dmas.append(softmax_rows(pl.ds(row, tc)))

        for dma in out_dmas:
            dma.wait()
        for c in range(N_CHUNKS):
            yrdmas[c].wait_send()
            xrdmas[c].wait_send()

    hbm = pltpu.MemorySpace.HBM
    return pl.pallas_call(
        body,
        out_shape=jax.ShapeDtypeStruct((t, v_global), jnp.bfloat16),
        in_specs=[
            pl.BlockSpec(memory_space=hbm),
            pl.BlockSpec(memory_space=hbm),
        ],
        out_specs=pl.BlockSpec(memory_space=hbm),
        scratch_shapes=[
            pltpu.VMEM((t, d), jnp.float32),
            pltpu.VMEM((d, v_local), jnp.float32),
            pltpu.VMEM((t, v_global), jnp.bfloat16),
            pltpu.VMEM((t, v_global), jnp.bfloat16),
            pltpu.SemaphoreType.DMA((2,)),
            pltpu.SemaphoreType.DMA,
            pltpu.SemaphoreType.DMA((N_CHUNKS,)),
            pltpu.SemaphoreType.DMA((N_CHUNKS,)),
            pltpu.SemaphoreType.DMA((N_CHUNKS,)),
            pltpu.SemaphoreType.DMA((N_CHUNKS,)),
        ],
        compiler_params=pltpu.CompilerParams(collective_id=0),
    )(x, W)


# device time: 26452 ns/iter; 1.0480x vs baseline; 1.0249x over previous
import jax
import jax.numpy as jnp
from jax import lax
from jax.experimental import pallas as pl
from jax.experimental.pallas import tpu as pltpu

N_CHUNKS = 8
GEMM_BLOCK = 64
CHUNKS_PER_BLOCK = GEMM_BLOCK // (128 // N_CHUNKS)


def kernel(x, W):
    t, d = x.shape
    _, v_local = W.shape
    v_global = 2 * v_local
    th = t // 2
    tc = th // N_CHUNKS
    n_blocks = th // GEMM_BLOCK

    def body(x_ref, w_ref, out_ref, log_ref,
             ysend_sems, yrecv_sems, xsend_sems, xrecv_sems):
        my_x = lax.axis_index("x")
        my_y = lax.axis_index("y")
        ypeer = (my_x, 1 - my_y)
        xpeer = (1 - my_x, my_y)

        barrier_sem = pltpu.get_barrier_semaphore()
        for nbr in (ypeer, xpeer):
            pl.semaphore_signal(
                barrier_sem, inc=1,
                device_id=nbr, device_id_type=pl.DeviceIdType.MESH,
            )

        w_bf16 = w_ref[...].astype(jnp.bfloat16)
        my_col = my_y * v_local
        peer_col = (1 - my_y) * v_local

        def gemm_block(row_start):
            log_ref[pl.ds(row_start, GEMM_BLOCK), pl.ds(my_col, v_local)] = (
                lax.dot_general(
                    x_ref[pl.ds(row_start, GEMM_BLOCK)].astype(jnp.bfloat16),
                    w_bf16,
                    (((1,), (0,)), ((), ())),
                    preferred_element_type=jnp.float32,
                ).astype(jnp.bfloat16)
            )

        def softmax_rows(rows):
            l0 = log_ref[rows, :v_local].astype(jnp.float32)
            l1 = log_ref[rows, v_local:].astype(jnp.float32)
            m = jnp.maximum(
                jnp.max(l0, axis=-1, keepdims=True),
                jnp.max(l1, axis=-1, keepdims=True),
            )
            e0 = jnp.exp(l0 - m)
            e1 = jnp.exp(l1 - m)
            s = jnp.sum(e0, axis=-1, keepdims=True) + jnp.sum(
                e1, axis=-1, keepdims=True
            )
            r = 1.0 / s
            out_ref[rows, :v_local] = (e0 * r).astype(jnp.bfloat16)
            out_ref[rows, v_local:] = (e1 * r).astype(jnp.bfloat16)

        yrdmas = []
        for b in range(n_blocks):
            gemm_block(my_x * th + b * GEMM_BLOCK)
            if b == 0:
                pl.semaphore_wait(barrier_sem, 2)
            for i in range(CHUNKS_PER_BLOCK):
                c = b * CHUNKS_PER_BLOCK + i
                row = my_x * th + c * tc
                rdma = pltpu.make_async_remote_copy(
                    src_ref=log_ref.at[pl.ds(row, tc), pl.ds(my_col, v_local)],
                    dst_ref=log_ref.at[pl.ds(row, tc), pl.ds(my_col, v_local)],
                    send_sem=ysend_sems.at[c],
                    recv_sem=yrecv_sems.at[c],
                    device_id=ypeer,
                    device_id_type=pl.DeviceIdType.MESH,
                )
                rdma.start()
                yrdmas.append(rdma)

        xrdmas = []
        for c in range(N_CHUNKS):
            row = my_x * th + c * tc
            rows = pl.ds(row, tc)
            yrdmas[c].wait_recv()
            rdma = pltpu.make_async_remote_copy(
                src_ref=log_ref.at[rows, pl.ds(peer_col, v_local)],
                dst_ref=log_ref.at[rows, pl.ds(peer_col, v_local)],
                send_sem=xsend_sems.at[c],
                recv_sem=xrecv_sems.at[c],
                device_id=xpeer,
                device_id_type=pl.DeviceIdType.MESH,
            )
            rdma.start()
            xrdmas.append(rdma)
            if c < n_blocks:
                gemm_block((1 - my_x) * th + c * GEMM_BLOCK)
            softmax_rows(rows)

        for c in range(N_CHUNKS):
            row = (1 - my_x) * th + c * tc
            xrdmas[c].wait_recv()
            softmax_rows(pl.ds(row, tc))

        for c in range(N_CHUNKS):
            yrdmas[c].wait_send()
            xrdmas[c].wait_send()

    return pl.pallas_call(
        body,
        out_shape=jax.ShapeDtypeStruct((t, v_global), jnp.bfloat16),
        in_specs=[
            pl.BlockSpec(memory_space=pltpu.VMEM),
            pl.BlockSpec(memory_space=pltpu.VMEM),
        ],
        out_specs=pl.BlockSpec(memory_space=pltpu.VMEM),
        scratch_shapes=[
            pltpu.VMEM((t, v_global), jnp.bfloat16),
            pltpu.SemaphoreType.DMA((N_CHUNKS,)),
            pltpu.SemaphoreType.DMA((N_CHUNKS,)),
            pltpu.SemaphoreType.DMA((N_CHUNKS,)),
            pltpu.SemaphoreType.DMA((N_CHUNKS,)),
        ],
        compiler_params=pltpu.CompilerParams(collective_id=0),
    )(x, W)
